# Initial kernel scaffold; baseline (speedup 1.0000x reference)
import jax
import jax.numpy as jnp
from jax.experimental import pallas as pl


def kernel(pos, edge_index, atom_types, type_embeddings):
    raise NotImplementedError("write your pallas kernel here")


# speedup vs baseline: 7.3279x; 7.3279x over previous
import functools

import jax
import jax.numpy as jnp
from jax import lax
from jax.experimental import pallas as pl
from jax.experimental.pallas import tpu as pltpu
from jax.experimental.pallas import tpu_sc as plsc

_N = 50000
_E = 1600000
_NUM_TYPES = 16
_NUM_BASIS = 8
_R_MAX = 4.0

_NSEG = 16
_EPS = _E // _NSEG
_C = 4000
_NCH = _EPS // _C
_IT = _C // 16

_G = 40
_BE = 128 * _G
_ROWS = _E // 128
_GRID = -(-_ROWS // _G)


def _sc_body(x_hbm, y_hbm, z_hbm, t_hbm, cidx_hbm, nidx_hbm,
             sxy_hbm, dz2_hbm, tcf_hbm, tnf_hbm,
             tab_a, tab_b, civ, niv, ob1, ob2, ob3):
    wid = lax.axis_index("s") * 2 + lax.axis_index("c")
    role = wid % 2
    seg = wid // 2
    base = seg * _EPS

    @pl.when(role == 0)
    def _():
        pltpu.sync_copy(x_hbm, tab_a)
        pltpu.sync_copy(y_hbm, tab_b)

    @pl.when(role == 1)
    def _():
        pltpu.sync_copy(z_hbm, tab_a)
        pltpu.sync_copy(t_hbm, tab_b)

    for ch in range(_NCH):
        off = base + ch * _C
        pltpu.sync_copy(cidx_hbm.at[pl.ds(off, _C)], civ)
        pltpu.sync_copy(nidx_hbm.at[pl.ds(off, _C)], niv)

        @pl.when(role == 0)
        def _():
            def body(i, carry):
                ci = civ[pl.ds(i * 16, 16)]
                ni = niv[pl.ds(i * 16, 16)]
                dx = (plsc.load_gather(tab_a, [ni])
                      - plsc.load_gather(tab_a, [ci]))
                dy = (plsc.load_gather(tab_b, [ni])
                      - plsc.load_gather(tab_b, [ci]))
                ob1[pl.ds(i * 16, 16)] = dx * dx + dy * dy
                return carry
            lax.fori_loop(0, _IT, body, 0)
            pltpu.sync_copy(ob1, sxy_hbm.at[pl.ds(off, _C)])

        @pl.when(role == 1)
        def _():
            def body(i, carry):
                ci = civ[pl.ds(i * 16, 16)]
                ni = niv[pl.ds(i * 16, 16)]
                dz = (plsc.load_gather(tab_a, [ni])
                      - plsc.load_gather(tab_a, [ci]))
                ob1[pl.ds(i * 16, 16)] = dz * dz
                ob2[pl.ds(i * 16, 16)] = plsc.load_gather(tab_b, [ci])
                ob3[pl.ds(i * 16, 16)] = plsc.load_gather(tab_b, [ni])
                return carry
            lax.fori_loop(0, _IT, body, 0)
            pltpu.sync_copy(ob1, dz2_hbm.at[pl.ds(off, _C)])
            pltpu.sync_copy(ob2, tcf_hbm.at[pl.ds(off, _C)])
            pltpu.sync_copy(ob3, tnf_hbm.at[pl.ds(off, _C)])


def _sc_gather(xs, ys, zs, ts, cidx, nidx):
    mesh = plsc.VectorSubcoreMesh(core_axis_name="c", subcore_axis_name="s")
    f32 = jnp.float32
    k = functools.partial(
        pl.kernel,
        mesh=mesh,
        out_type=[jax.ShapeDtypeStruct((_E,), f32)] * 4,
        compiler_params=pltpu.CompilerParams(needs_layout_passes=False),
        scratch_types=[
            pltpu.VMEM((_N,), f32),
            pltpu.VMEM((_N,), f32),
            pltpu.VMEM((_C,), jnp.int32),
            pltpu.VMEM((_C,), jnp.int32),
            pltpu.VMEM((_C,), f32),
            pltpu.VMEM((_C,), f32),
            pltpu.VMEM((_C,), f32),
        ],
    )(_sc_body)
    return k(xs, ys, zs, ts, cidx, nidx)


def _tc_body(sxy_ref, dz2_ref, tcf_ref, tnf_ref, eye_ref, w_ref, out_ref):
    i = pl.program_id(0)
    riota = lax.broadcasted_iota(jnp.int32, (_G, 1), 0)
    valid = (i * _G + riota) < _ROWS
    r2 = jnp.where(valid, sxy_ref[...] + dz2_ref[...], 1.0)
    tcf = jnp.where(valid, tcf_ref[...], 0.0)
    tnf = jnp.where(valid, tnf_ref[...], 0.0)
    a = jnp.concatenate([r2, tcf, tnf], axis=0)
    t = lax.dot_general(a, eye_ref[...],
                        dimension_numbers=(((0,), (0,)), ((), ())),
                        preferred_element_type=jnp.float32)
    karr = (lax.broadcasted_iota(jnp.int32, (1, _NUM_BASIS), 1) + 1
            ).astype(jnp.float32)
    tlan = lax.broadcasted_iota(jnp.int32, (1, _NUM_TYPES), 1).astype(jnp.float32)
    pref = jnp.float32(0.7071067811865476)
    for g in range(_G):
        r2c = t[:, g:g + 1]
        tcc = t[:, _G + g:_G + g + 1]
        tnc = t[:, 2 * _G + g:2 * _G + g + 1]
        rinv = lax.rsqrt(r2c)
        r = r2c * rinv
        u = r * (1.0 / _R_MAX)
        u2 = u * u
        u6 = u2 * u2 * u2
        cut = 1.0 - 28.0 * u6 + 48.0 * u6 * u - 21.0 * u6 * u2
        cut = jnp.where(u < 1.0, cut, 0.0)
        theta = (jnp.pi / _R_MAX) * r
        basis = pref * jnp.sin(theta * karr) * rinv
        oh = jnp.concatenate([(tcc == tlan).astype(jnp.float32),
                              (tnc == tlan).astype(jnp.float32)], axis=1)
        emb = jnp.dot(oh, w_ref[...], preferred_element_type=jnp.float32)
        row = jnp.concatenate([basis, emb], axis=1) * cut
        out_ref[pl.ds(g * 128, 128), :] = row


def _tc_compute(sxy, dz2, tcf, tnf, eye3g, w):
    f32 = jnp.float32
    lane_spec = pl.BlockSpec((_G, 128), lambda i: (i, 0))
    return pl.pallas_call(
        _tc_body,
        grid=(_GRID,),
        in_specs=[
            lane_spec, lane_spec, lane_spec, lane_spec,
            pl.BlockSpec((3 * _G, 3 * _G), lambda i: (0, 0)),
            pl.BlockSpec((2 * _NUM_TYPES, 2 * _NUM_TYPES), lambda i: (0, 0)),
        ],
        out_specs=pl.BlockSpec((_BE, _NUM_BASIS + 2 * _NUM_TYPES),
                               lambda i: (i, 0)),
        out_shape=jax.ShapeDtypeStruct((_E, _NUM_BASIS + 2 * _NUM_TYPES), f32),
    )(sxy, dz2, tcf, tnf, eye3g, w)


def kernel(pos, edge_index, atom_types, type_embeddings):
    f32 = jnp.float32
    xs = pos[:, 0].astype(f32)
    ys = pos[:, 1].astype(f32)
    zs = pos[:, 2].astype(f32)
    ts = atom_types.astype(f32)
    cidx = edge_index[0].astype(jnp.int32)
    nidx = edge_index[1].astype(jnp.int32)
    sxy, dz2, tcf, tnf = _sc_gather(xs, ys, zs, ts, cidx, nidx)
    sxy = sxy.reshape(_ROWS, 128)
    dz2 = dz2.reshape(_ROWS, 128)
    tcf = tcf.reshape(_ROWS, 128)
    tnf = tnf.reshape(_ROWS, 128)
    eye3g = jnp.eye(3 * _G, dtype=f32)
    emb0 = type_embeddings[0].astype(f32)
    emb1 = type_embeddings[1].astype(f32)
    zero = jnp.zeros((_NUM_TYPES, _NUM_TYPES), f32)
    w = jnp.block([[emb0, zero], [zero, emb1]])
    return _tc_compute(sxy, dz2, tcf, tnf, eye3g, w)


# speedup vs baseline: 10.4684x; 1.4286x over previous
import functools

import jax
import jax.numpy as jnp
from jax import lax
from jax.experimental import pallas as pl
from jax.experimental.pallas import tpu as pltpu
from jax.experimental.pallas import tpu_sc as plsc

_N = 50000
_E = 1600000
_NUM_TYPES = 16
_NUM_BASIS = 8
_R_MAX = 4.0

_NSEG = 16
_EPS = _E // _NSEG
_C = 4000
_NCH = _EPS // _C
_IT = _C // 16

_G = 40
_BE = 128 * _G
_ROWS = _E // 128
_GRID = -(-_ROWS // _G)


def _sc_body(x_hbm, y_hbm, z_hbm, t_hbm, cidx_hbm, nidx_hbm,
             sxy_hbm, dz2_hbm, tcf_hbm, tnf_hbm,
             tab_a, tab_b, civ, niv, ob1, ob2, ob3):
    wid = lax.axis_index("s") * 2 + lax.axis_index("c")
    role = wid % 2
    seg = wid // 2
    base = seg * _EPS

    @pl.when(role == 0)
    def _():
        pltpu.sync_copy(x_hbm, tab_a)
        pltpu.sync_copy(y_hbm, tab_b)

    @pl.when(role == 1)
    def _():
        pltpu.sync_copy(z_hbm, tab_a)
        pltpu.sync_copy(t_hbm, tab_b)

    for ch in range(_NCH):
        off = base + ch * _C
        pltpu.sync_copy(cidx_hbm.at[pl.ds(off, _C)], civ)
        pltpu.sync_copy(nidx_hbm.at[pl.ds(off, _C)], niv)

        @pl.when(role == 0)
        def _():
            def body(i, carry):
                ci = civ[pl.ds(i * 16, 16)]
                ni = niv[pl.ds(i * 16, 16)]
                dx = (plsc.load_gather(tab_a, [ni])
                      - plsc.load_gather(tab_a, [ci]))
                dy = (plsc.load_gather(tab_b, [ni])
                      - plsc.load_gather(tab_b, [ci]))
                ob1[pl.ds(i * 16, 16)] = dx * dx + dy * dy
                return carry
            lax.fori_loop(0, _IT, body, 0)
            pltpu.sync_copy(ob1, sxy_hbm.at[pl.ds(off, _C)])

        @pl.when(role == 1)
        def _():
            def body(i, carry):
                ci = civ[pl.ds(i * 16, 16)]
                ni = niv[pl.ds(i * 16, 16)]
                dz = (plsc.load_gather(tab_a, [ni])
                      - plsc.load_gather(tab_a, [ci]))
                ob1[pl.ds(i * 16, 16)] = dz * dz
                ob2[pl.ds(i * 16, 16)] = plsc.load_gather(tab_b, [ci])
                ob3[pl.ds(i * 16, 16)] = plsc.load_gather(tab_b, [ni])
                return carry
            lax.fori_loop(0, _IT, body, 0)
            pltpu.sync_copy(ob1, dz2_hbm.at[pl.ds(off, _C)])
            pltpu.sync_copy(ob2, tcf_hbm.at[pl.ds(off, _C)])
            pltpu.sync_copy(ob3, tnf_hbm.at[pl.ds(off, _C)])


def _sc_gather(xs, ys, zs, ts, cidx, nidx):
    mesh = plsc.VectorSubcoreMesh(core_axis_name="c", subcore_axis_name="s")
    f32 = jnp.float32
    k = functools.partial(
        pl.kernel,
        mesh=mesh,
        out_type=[jax.ShapeDtypeStruct((_E,), f32)] * 4,
        compiler_params=pltpu.CompilerParams(needs_layout_passes=False),
        scratch_types=[
            pltpu.VMEM((_N,), f32),
            pltpu.VMEM((_N,), f32),
            pltpu.VMEM((_C,), jnp.int32),
            pltpu.VMEM((_C,), jnp.int32),
            pltpu.VMEM((_C,), f32),
            pltpu.VMEM((_C,), f32),
            pltpu.VMEM((_C,), f32),
        ],
    )(_sc_body)
    return k(xs, ys, zs, ts, cidx, nidx)


_SIN_C = (9.99999946e-01, -1.66666446e-01, 8.33307884e-03,
          -1.98287406e-04, 2.72576489e-06, -2.15868285e-08)
_COS_C = (9.99999999e-01, -4.99999985e-01, 4.16666317e-02,
          -1.38885881e-03, 2.47893192e-05, -2.73004342e-07,
          1.81904403e-09)


def _poly_even(x2, coefs):
    acc = jnp.float32(coefs[-1])
    for c in coefs[-2::-1]:
        acc = acc * x2 + jnp.float32(c)
    return acc


def _tc_body(sxy_ref, dz2_ref, tcf_ref, tnf_ref, perm_ref, w_ref, out_ref):
    i = pl.program_id(0)
    riota = lax.broadcasted_iota(jnp.int32, (_G, 1), 0)
    valid = (i * _G + riota) < _ROWS
    r2 = jnp.where(valid, sxy_ref[...] + dz2_ref[...], 1.0)
    tcf = jnp.where(valid, tcf_ref[...], 0.0)
    tnf = jnp.where(valid, tnf_ref[...], 0.0)
    rinv = lax.rsqrt(r2)
    r = r2 * rinv
    u = r * (1.0 / _R_MAX)
    u2 = u * u
    u6 = u2 * u2 * u2
    cut = 1.0 - 28.0 * u6 + 48.0 * u6 * u - 21.0 * u6 * u2
    cut = jnp.where(u < 1.0, cut, 0.0)
    th = (jnp.pi / _R_MAX) * r
    th2 = th * th
    s1 = th * _poly_even(th2, _SIN_C)
    c2 = _poly_even(th2, _COS_C) * 2.0
    m = jnp.float32(0.7071067811865476) * rinv * cut
    fields = [m * s1]
    sprev, scur = jnp.zeros_like(s1), s1
    for _ in range(_NUM_BASIS - 1):
        sprev, scur = scur, c2 * scur - sprev
        fields.append(m * scur)
    fields += [tcf, tnf, cut]
    a = jnp.concatenate(fields, axis=0)
    t = lax.dot_general(a, perm_ref[...],
                        dimension_numbers=(((0,), (0,)), ((), ())),
                        preferred_element_type=jnp.float32)
    lane32 = lax.broadcasted_iota(jnp.int32, (1, 2 * _NUM_TYPES), 1)
    iotam = (lane32 % _NUM_TYPES).astype(jnp.float32)
    half = lane32 < _NUM_TYPES
    for g in range(_G):
        blk = t[:, 11 * g:11 * g + 11]
        basis = blk[:, 0:_NUM_BASIS]
        tcc = blk[:, 8:9]
        tnc = blk[:, 9:10]
        cutc = blk[:, 10:11]
        col32 = jnp.where(half, tcc, tnc)
        oh = jnp.where(col32 == iotam, cutc, 0.0)
        emb = jnp.dot(oh, w_ref[...], preferred_element_type=jnp.float32)
        out_ref[pl.ds(g * 128, 128), :] = jnp.concatenate([basis, emb], axis=1)


def _tc_compute(sxy, dz2, tcf, tnf, perm, w):
    f32 = jnp.float32
    lane_spec = pl.BlockSpec((_G, 128), lambda i: (i, 0))
    return pl.pallas_call(
        _tc_body,
        grid=(_GRID,),
        in_specs=[
            lane_spec, lane_spec, lane_spec, lane_spec,
            pl.BlockSpec((11 * _G, 11 * _G), lambda i: (0, 0)),
            pl.BlockSpec((2 * _NUM_TYPES, 2 * _NUM_TYPES), lambda i: (0, 0)),
        ],
        out_specs=pl.BlockSpec((_BE, _NUM_BASIS + 2 * _NUM_TYPES),
                               lambda i: (i, 0)),
        out_shape=jax.ShapeDtypeStruct((_E, _NUM_BASIS + 2 * _NUM_TYPES), f32),
    )(sxy, dz2, tcf, tnf, perm, w)


def kernel(pos, edge_index, atom_types, type_embeddings):
    f32 = jnp.float32
    xs = pos[:, 0].astype(f32)
    ys = pos[:, 1].astype(f32)
    zs = pos[:, 2].astype(f32)
    ts = atom_types.astype(f32)
    cidx = edge_index[0].astype(jnp.int32)
    nidx = edge_index[1].astype(jnp.int32)
    sxy, dz2, tcf, tnf = _sc_gather(xs, ys, zs, ts, cidx, nidx)
    sxy = sxy.reshape(_ROWS, 128)
    dz2 = dz2.reshape(_ROWS, 128)
    tcf = tcf.reshape(_ROWS, 128)
    tnf = tnf.reshape(_ROWS, 128)
    src = jnp.arange(11 * _G)
    f_idx, g_idx = src // _G, src % _G
    perm = jnp.zeros((11 * _G, 11 * _G), f32).at[src, 11 * g_idx + f_idx].set(1.0)
    emb0 = type_embeddings[0].astype(f32)
    emb1 = type_embeddings[1].astype(f32)
    zero = jnp.zeros((_NUM_TYPES, _NUM_TYPES), f32)
    w = jnp.block([[emb0, zero], [zero, emb1]])
    return _tc_compute(sxy, dz2, tcf, tnf, perm, w)


# speedup vs baseline: 32.9612x; 3.1486x over previous
import functools

import jax
import jax.numpy as jnp
from jax import lax
from jax.experimental import pallas as pl
from jax.experimental.pallas import tpu as pltpu
from jax.experimental.pallas import tpu_sc as plsc

_N = 50000
_E = 1600000
_NUM_TYPES = 16
_NUM_BASIS = 8
_R_MAX = 4.0

_NSEG = 16
_EPS = _E // _NSEG
_C = 4000
_NCH = _EPS // _C
_IT = _C // 16

_G = 40
_BE = 128 * _G
_ROWS = _E // 128
_GRID = -(-_ROWS // _G)


def _sc_body(x_hbm, y_hbm, z_hbm, t_hbm, cidx_hbm, nidx_hbm,
             sxy_hbm, dz2_hbm, tcf_hbm, tnf_hbm,
             tab_a, tab_b, civ, niv, ob1, ob2, ob3):
    wid = lax.axis_index("s") * 2 + lax.axis_index("c")
    role = wid % 2
    seg = wid // 2
    base = seg * _EPS

    @pl.when(role == 0)
    def _():
        pltpu.sync_copy(x_hbm, tab_a)
        pltpu.sync_copy(y_hbm, tab_b)

    @pl.when(role == 1)
    def _():
        pltpu.sync_copy(z_hbm, tab_a)
        pltpu.sync_copy(t_hbm, tab_b)

    for ch in range(_NCH):
        off = base + ch * _C
        pltpu.sync_copy(cidx_hbm.at[pl.ds(off, _C)], civ)
        pltpu.sync_copy(nidx_hbm.at[pl.ds(off, _C)], niv)

        @pl.when(role == 0)
        def _():
            def body(i, carry):
                ci = civ[pl.ds(i * 16, 16)]
                ni = niv[pl.ds(i * 16, 16)]
                dx = (plsc.load_gather(tab_a, [ni])
                      - plsc.load_gather(tab_a, [ci]))
                dy = (plsc.load_gather(tab_b, [ni])
                      - plsc.load_gather(tab_b, [ci]))
                ob1[pl.ds(i * 16, 16)] = dx * dx + dy * dy
                return carry
            lax.fori_loop(0, _IT, body, 0)
            pltpu.sync_copy(ob1, sxy_hbm.at[pl.ds(off, _C)])

        @pl.when(role == 1)
        def _():
            def body(i, carry):
                ci = civ[pl.ds(i * 16, 16)]
                ni = niv[pl.ds(i * 16, 16)]
                dz = (plsc.load_gather(tab_a, [ni])
                      - plsc.load_gather(tab_a, [ci]))
                ob1[pl.ds(i * 16, 16)] = dz * dz
                ob2[pl.ds(i * 16, 16)] = plsc.load_gather(tab_b, [ci])
                ob3[pl.ds(i * 16, 16)] = plsc.load_gather(tab_b, [ni])
                return carry
            lax.fori_loop(0, _IT, body, 0)
            pltpu.sync_copy(ob1, dz2_hbm.at[pl.ds(off, _C)])
            pltpu.sync_copy(ob2, tcf_hbm.at[pl.ds(off, _C)])
            pltpu.sync_copy(ob3, tnf_hbm.at[pl.ds(off, _C)])


def _sc_gather(xs, ys, zs, ts, cidx, nidx):
    mesh = plsc.VectorSubcoreMesh(core_axis_name="c", subcore_axis_name="s")
    f32 = jnp.float32
    k = functools.partial(
        pl.kernel,
        mesh=mesh,
        out_type=[jax.ShapeDtypeStruct((_E,), f32)] * 4,
        compiler_params=pltpu.CompilerParams(needs_layout_passes=False),
        scratch_types=[
            pltpu.VMEM((_N,), f32),
            pltpu.VMEM((_N,), f32),
            pltpu.VMEM((_C,), jnp.int32),
            pltpu.VMEM((_C,), jnp.int32),
            pltpu.VMEM((_C,), f32),
            pltpu.VMEM((_C,), f32),
            pltpu.VMEM((_C,), f32),
        ],
    )(_sc_body)
    return k(xs, ys, zs, ts, cidx, nidx)


_SIN_C = (9.99999946e-01, -1.66666446e-01, 8.33307884e-03,
          -1.98287406e-04, 2.72576489e-06, -2.15868285e-08)
_COS_C = (9.99999999e-01, -4.99999985e-01, 4.16666317e-02,
          -1.38885881e-03, 2.47893192e-05, -2.73004342e-07,
          1.81904403e-09)


def _poly_even(x2, coefs):
    acc = jnp.float32(coefs[-1])
    for c in coefs[-2::-1]:
        acc = acc * x2 + jnp.float32(c)
    return acc


_SUB = 8
_NSUB = _G // _SUB
_NF = _NUM_BASIS + 2 * _NUM_TYPES


def _tc_body(sxy_ref, dz2_ref, tcf_ref, tnf_ref, r_ref, out_ref):
    i = pl.program_id(0)
    riota = lax.broadcasted_iota(jnp.int32, (_G, 1), 0)
    valid = (i * _G + riota) < _ROWS
    r2 = jnp.where(valid, sxy_ref[...] + dz2_ref[...], 1.0)
    tcf = jnp.where(valid, tcf_ref[...], 0.0)
    tnf = jnp.where(valid, tnf_ref[...], 0.0)
    rinv = lax.rsqrt(r2)
    r = r2 * rinv
    u = r * (1.0 / _R_MAX)
    u2 = u * u
    u6 = u2 * u2 * u2
    cut = 1.0 - 28.0 * u6 + 48.0 * u6 * u - 21.0 * u6 * u2
    cut = jnp.where(u < 1.0, cut, 0.0)
    th = (jnp.pi / _R_MAX) * r
    th2 = th * th
    s1 = th * _poly_even(th2, _SIN_C)
    c2 = _poly_even(th2, _COS_C) * 2.0
    m = jnp.float32(0.7071067811865476) * rinv * cut
    fields = [m * s1]
    sprev, scur = jnp.zeros_like(s1), s1
    for _ in range(_NUM_BASIS - 1):
        sprev, scur = scur, c2 * scur - sprev
        fields.append(m * scur)
    for t in range(_NUM_TYPES):
        fields.append(jnp.where(tcf == float(t), cut, 0.0))
    for t in range(_NUM_TYPES):
        fields.append(jnp.where(tnf == float(t), cut, 0.0))
    bf = [f.astype(jnp.bfloat16) for f in fields]
    for s in range(_NSUB):
        a_s = jnp.concatenate([f[_SUB * s:_SUB * (s + 1)] for f in bf],
                              axis=0)
        t_s = lax.dot_general(a_s, r_ref[...],
                              dimension_numbers=(((0,), (0,)), ((), ())),
                              preferred_element_type=jnp.float32)
        for gg in range(_SUB):
            out_ref[pl.ds((_SUB * s + gg) * 128, 128), :] = (
                t_s[:, 128 * gg:128 * gg + _NF])


def _tc_compute(sxy, dz2, tcf, tnf, rmat):
    f32 = jnp.float32
    lane_spec = pl.BlockSpec((_G, 128), lambda i: (i, 0))
    return pl.pallas_call(
        _tc_body,
        grid=(_GRID,),
        in_specs=[
            lane_spec, lane_spec, lane_spec, lane_spec,
            pl.BlockSpec((_NF * _SUB, 128 * _SUB), lambda i: (0, 0)),
        ],
        out_specs=pl.BlockSpec((_BE, _NF), lambda i: (i, 0)),
        out_shape=jax.ShapeDtypeStruct((_E, _NF), f32),
    )(sxy, dz2, tcf, tnf, rmat)


def kernel(pos, edge_index, atom_types, type_embeddings):
    f32 = jnp.float32
    xs = pos[:, 0].astype(f32)
    ys = pos[:, 1].astype(f32)
    zs = pos[:, 2].astype(f32)
    ts = atom_types.astype(f32)
    cidx = edge_index[0].astype(jnp.int32)
    nidx = edge_index[1].astype(jnp.int32)
    sxy, dz2, tcf, tnf = _sc_gather(xs, ys, zs, ts, cidx, nidx)
    sxy = sxy.reshape(_ROWS, 128)
    dz2 = dz2.reshape(_ROWS, 128)
    tcf = tcf.reshape(_ROWS, 128)
    tnf = tnf.reshape(_ROWS, 128)
    return _tc_compute(sxy, dz2, tcf, tnf, _build_rmat(type_embeddings))


def _build_rmat(type_embeddings):
    f32 = jnp.float32
    emb0 = type_embeddings[0].astype(f32)
    emb1 = type_embeddings[1].astype(f32)
    rmat = jnp.zeros((_NF * _SUB, 128 * _SUB), f32)
    rr = jnp.arange(_SUB)
    jb = jnp.arange(_NUM_BASIS)
    je = jnp.arange(_NUM_TYPES)
    tt = jnp.arange(_NUM_TYPES)
    rows = (jb[:, None] * _SUB + rr[None, :])
    cols = (128 * rr[None, :] + jb[:, None])
    rmat = rmat.at[rows, cols].set(1.0)
    rows = ((_NUM_BASIS + tt)[:, None, None] * _SUB + rr[None, :, None])
    cols = (128 * rr[None, :, None] + _NUM_BASIS + je[None, None, :])
    rmat = rmat.at[jnp.broadcast_to(rows, (16, _SUB, 16)),
                   jnp.broadcast_to(cols, (16, _SUB, 16))].set(
        jnp.broadcast_to(emb0[:, None, :], (16, _SUB, 16)))
    rows = ((_NUM_BASIS + _NUM_TYPES + tt)[:, None, None] * _SUB
            + rr[None, :, None])
    cols = (128 * rr[None, :, None] + _NUM_BASIS + _NUM_TYPES
            + je[None, None, :])
    rmat = rmat.at[jnp.broadcast_to(rows, (16, _SUB, 16)),
                   jnp.broadcast_to(cols, (16, _SUB, 16))].set(
        jnp.broadcast_to(emb1[:, None, :], (16, _SUB, 16)))
    return rmat.astype(jnp.bfloat16)
